# Initial kernel scaffold; baseline (speedup 1.0000x reference)
import jax
import jax.numpy as jnp
from jax.experimental import pallas as pl


def kernel(x, mask):
    raise NotImplementedError("write your pallas kernel here")


# speedup vs baseline: 3.3179x; 3.3179x over previous
import functools

import jax
import jax.numpy as jnp
from jax import lax
from jax.experimental import pallas as pl
from jax.experimental.pallas import tpu as pltpu
from jax.experimental.pallas import tpu_sc as plsc

B = 64
V = 4096
N = 512 * 512

NC = 2
NS = 16
NW = NC * NS

BG = 8
BPG = B // BG
IC = NW // BG
IPC = N // IC

W = 2048
NSTEP = IPC // W
LANES = 16


def _sc_gather(x_flat, mask_flat):
    mesh = plsc.VectorSubcoreMesh(core_axis_name="c", subcore_axis_name="s")

    @functools.partial(
        pl.kernel,
        mesh=mesh,
        compiler_params=pltpu.CompilerParams(needs_layout_passes=False),
        out_type=jax.ShapeDtypeStruct((B, N), jnp.float32),
        scratch_types=[
            pltpu.VMEM((BPG * V,), jnp.float32),
            pltpu.VMEM((W,), jnp.int32),
            pltpu.VMEM((BPG, W), jnp.float32),
        ],
    )
    def k(x_hbm, mask_hbm, out_hbm, x_v, idx_v, out_v):
        wid = lax.axis_index("s") * NC + lax.axis_index("c")
        g = wid // IC
        c = wid % IC
        row0 = g * BPG
        pltpu.sync_copy(x_hbm.at[pl.ds(row0 * V, BPG * V)], x_v)

        def step(ci, carry):
            base = c * IPC + ci * W
            pltpu.sync_copy(mask_hbm.at[pl.ds(base, W)], idx_v)

            def vbody(i, carry2):
                idx = idx_v[pl.ds(i * LANES, LANES)]
                for b in range(BPG):
                    vals = plsc.load_gather(x_v, [idx + b * V])
                    out_v[b, pl.ds(i * LANES, LANES)] = vals
                return carry2

            lax.fori_loop(0, W // LANES, vbody, 0)
            for b in range(BPG):
                pltpu.sync_copy(out_v.at[b], out_hbm.at[row0 + b, pl.ds(base, W)])
            return carry

        lax.fori_loop(0, NSTEP, step, 0)

    return k(x_flat, mask_flat)


def kernel(x, mask):
    out = _sc_gather(x.reshape(-1), mask.reshape(-1))
    return out.reshape(x.shape[0], *mask.shape)


# speedup vs baseline: 3.9194x; 1.1813x over previous
import functools

import jax
import jax.numpy as jnp
from jax import lax
from jax.experimental import pallas as pl
from jax.experimental.pallas import tpu as pltpu
from jax.experimental.pallas import tpu_sc as plsc

B = 64
V = 4096
N = 512 * 512

NC = 2
NS = 16
NW = NC * NS

BG = 8
BPG = B // BG
IC = NW // BG
IPC = N // IC

W = 2048
NSTEP = IPC // W
LANES = 16


def _sc_gather(x_flat, mask_flat):
    mesh = plsc.VectorSubcoreMesh(core_axis_name="c", subcore_axis_name="s")

    @functools.partial(
        pl.kernel,
        mesh=mesh,
        compiler_params=pltpu.CompilerParams(needs_layout_passes=False),
        out_type=jax.ShapeDtypeStruct((B, N), jnp.float32),
        scratch_types=[
            pltpu.VMEM((BPG * V,), jnp.float32),
            pltpu.VMEM((W,), jnp.int32),
            pltpu.VMEM((W,), jnp.int32),
            pltpu.VMEM((BPG, W), jnp.float32),
            pltpu.VMEM((BPG, W), jnp.float32),
            pltpu.SemaphoreType.DMA,
            pltpu.SemaphoreType.DMA,
            pltpu.SemaphoreType.DMA,
            pltpu.SemaphoreType.DMA,
        ],
    )
    def k(x_hbm, mask_hbm, out_hbm, x_v, idx0, idx1, out0, out1,
          si0, si1, so0, so1):
        wid = lax.axis_index("s") * NC + lax.axis_index("c")
        g = wid // IC
        c = wid % IC
        row0 = g * BPG
        cbase = c * IPC
        pltpu.sync_copy(x_hbm.at[pl.ds(row0 * V, BPG * V)], x_v)

        idxb = (idx0, idx1)
        outb = (out0, out1)
        sib = (si0, si1)
        sob = (so0, so1)

        def in_copy(s, par):
            return pltpu.make_async_copy(
                mask_hbm.at[pl.ds(cbase + s * W, W)], idxb[par], sib[par])

        def out_copy(base, par, b):
            return pltpu.make_async_copy(
                outb[par].at[b], out_hbm.at[row0 + b, pl.ds(base, W)],
                sob[par])

        in_copy(0, 0).start()
        in_copy(1, 1).start()

        def step2(i, carry):
            for par in range(2):
                s = 2 * i + par
                base = cbase + s * W
                in_copy(s, par).wait()

                @pl.when(s >= 2)
                def _drain():
                    for b in range(BPG):
                        out_copy(cbase, par, b).wait()

                def vbody(v, carry2):
                    idx = idxb[par][pl.ds(v * LANES, LANES)]
                    for b in range(BPG):
                        outb[par][b, pl.ds(v * LANES, LANES)] = (
                            plsc.load_gather(x_v, [idx + b * V]))
                    return carry2

                lax.fori_loop(0, W // LANES, vbody, 0, unroll=4)

                @pl.when(s + 2 < NSTEP)
                def _prefetch():
                    in_copy(s + 2, par).start()

                for b in range(BPG):
                    out_copy(base, par, b).start()
            return carry

        lax.fori_loop(0, NSTEP // 2, step2, 0)
        for par in range(2):
            for b in range(BPG):
                out_copy(cbase, par, b).wait()

    return k(x_flat, mask_flat)


def kernel(x, mask):
    out = _sc_gather(x.reshape(-1), mask.reshape(-1))
    return out.reshape(x.shape[0], *mask.shape)


# speedup vs baseline: 8.0164x; 2.0453x over previous
import functools

import jax
import jax.numpy as jnp
from jax import lax
from jax.experimental import pallas as pl
from jax.experimental.pallas import tpu as pltpu
from jax.experimental.pallas import tpu_sc as plsc

B = 64
V = 4096
N = 512 * 512

NC = 2
NS = 16
NW = NC * NS

BG = 8
BPG = B // BG
IC = NW // BG
IPC = N // IC

W = 2048
NSTEP = IPC // W
LANES = 16


def _sc_gather(x_flat, mask_flat):
    mesh = plsc.VectorSubcoreMesh(core_axis_name="c", subcore_axis_name="s")

    @functools.partial(
        pl.kernel,
        mesh=mesh,
        compiler_params=pltpu.CompilerParams(needs_layout_passes=False),
        out_type=jax.ShapeDtypeStruct((B, N), jnp.float32),
        scratch_types=[
            pltpu.VMEM((BPG * V,), jnp.float32),
            pltpu.VMEM((W,), jnp.int32),
            pltpu.VMEM((W,), jnp.int32),
            pltpu.VMEM((BPG, W), jnp.float32),
            pltpu.VMEM((BPG, W), jnp.float32),
            pltpu.SemaphoreType.DMA,
            pltpu.SemaphoreType.DMA,
            pltpu.SemaphoreType.DMA,
            pltpu.SemaphoreType.DMA,
        ],
    )
    def k(x_hbm, mask_hbm, out_hbm, x_v, idx0, idx1, out0, out1,
          si0, si1, so0, so1):
        wid = lax.axis_index("s") * NC + lax.axis_index("c")
        g = wid // IC
        c = wid % IC
        row0 = g * BPG
        cbase = c * IPC
        pltpu.sync_copy(x_hbm.at[pl.ds(row0 * V, BPG * V)], x_v)

        idxb = (idx0, idx1)
        outb = (out0, out1)
        sib = (si0, si1)
        sob = (so0, so1)

        def in_copy(s, par):
            return pltpu.make_async_copy(
                mask_hbm.at[pl.ds(cbase + s * W, W)], idxb[par], sib[par])

        def out_copy(base, par, b):
            return pltpu.make_async_copy(
                outb[par].at[b], out_hbm.at[row0 + b, pl.ds(base, W)],
                sob[par])

        in_copy(0, 0).start()
        in_copy(1, 1).start()

        def step2(i, carry):
            for par in range(2):
                s = 2 * i + par
                base = cbase + s * W
                in_copy(s, par).wait()

                @pl.when(s >= 2)
                def _drain():
                    for b in range(BPG):
                        out_copy(cbase, par, b).wait()

                @plsc.parallel_loop(0, W // LANES, unroll=4)
                def _vbody(v):
                    idx = idxb[par][pl.ds(v * LANES, LANES)]
                    for b in range(BPG):
                        outb[par][b, pl.ds(v * LANES, LANES)] = (
                            plsc.load_gather(x_v, [idx + b * V]))

                @pl.when(s + 2 < NSTEP)
                def _prefetch():
                    in_copy(s + 2, par).start()

                for b in range(BPG):
                    out_copy(base, par, b).start()
            return carry

        lax.fori_loop(0, NSTEP // 2, step2, 0)
        for par in range(2):
            for b in range(BPG):
                out_copy(cbase, par, b).wait()

    return k(x_flat, mask_flat)


def kernel(x, mask):
    out = _sc_gather(x.reshape(-1), mask.reshape(-1))
    return out.reshape(x.shape[0], *mask.shape)


# speedup vs baseline: 13.2960x; 1.6586x over previous
import functools

import jax
import jax.numpy as jnp
from jax import lax
from jax.experimental import pallas as pl
from jax.experimental.pallas import tpu as pltpu
from jax.experimental.pallas import tpu_sc as plsc

B = 64
V = 4096
R = 512
C = 512

NC = 2
NS = 16
NW = NC * NS

BG = 8
BPG = B // BG
IC = NW // BG
NBR = R // 8
BRPC = NBR // IC

LANES = 16
JV = C // LANES


def _sc_gather(x_flat, mask):
    mesh = plsc.VectorSubcoreMesh(core_axis_name="c", subcore_axis_name="s")

    @functools.partial(
        pl.kernel,
        mesh=mesh,
        compiler_params=pltpu.CompilerParams(needs_layout_passes=False),
        out_type=jax.ShapeDtypeStruct((B, R, C), jnp.float32),
        scratch_types=[
            pltpu.VMEM((BPG * V,), jnp.float32),
            pltpu.VMEM((8, C), jnp.int32),
            pltpu.VMEM((8, C), jnp.int32),
            pltpu.VMEM((BPG, 8, C), jnp.float32),
            pltpu.VMEM((BPG, 8, C), jnp.float32),
            pltpu.SemaphoreType.DMA,
            pltpu.SemaphoreType.DMA,
            pltpu.SemaphoreType.DMA,
            pltpu.SemaphoreType.DMA,
        ],
    )
    def k(x_hbm, mask_hbm, out_hbm, x_v, idx0, idx1, out0, out1,
          si0, si1, so0, so1):
        wid = lax.axis_index("s") * NC + lax.axis_index("c")
        g = wid // IC
        c = wid % IC
        row0 = g * BPG
        br0 = c * BRPC
        pltpu.sync_copy(x_hbm.at[pl.ds(row0 * V, BPG * V)], x_v)

        idxb = (idx0, idx1)
        outb = (out0, out1)
        sib = (si0, si1)
        sob = (so0, so1)

        def in_copy(s, par):
            return pltpu.make_async_copy(
                mask_hbm.at[pl.ds((br0 + s) * 8, 8)], idxb[par], sib[par])

        def out_copy(s, par, b):
            return pltpu.make_async_copy(
                outb[par].at[b],
                out_hbm.at[row0 + b, pl.ds((br0 + s) * 8, 8)],
                sob[par])

        in_copy(0, 0).start()
        in_copy(1, 1).start()

        def step2(i, carry):
            for par in range(2):
                s = 2 * i + par
                in_copy(s, par).wait()

                @pl.when(s >= 2)
                def _drain():
                    for b in range(BPG):
                        out_copy(0, par, b).wait()

                for ii in range(8):
                    @plsc.parallel_loop(0, JV, unroll=4)
                    def _vbody(v):
                        idx = idxb[par][ii, pl.ds(v * LANES, LANES)]
                        for b in range(BPG):
                            outb[par][b, ii, pl.ds(v * LANES, LANES)] = (
                                plsc.load_gather(x_v, [idx + b * V]))

                @pl.when(s + 2 < BRPC)
                def _prefetch():
                    in_copy(s + 2, par).start()

                for b in range(BPG):
                    out_copy(s, par, b).start()
            return carry

        lax.fori_loop(0, BRPC // 2, step2, 0)
        for par in range(2):
            for b in range(BPG):
                out_copy(0, par, b).wait()

    return k(x_flat, mask)


def kernel(x, mask):
    return _sc_gather(x.reshape(-1), mask)


# speedup vs baseline: 15.3213x; 1.1523x over previous
import functools

import jax
import jax.numpy as jnp
from jax import lax
from jax.experimental import pallas as pl
from jax.experimental.pallas import tpu as pltpu
from jax.experimental.pallas import tpu_sc as plsc

B = 64
V = 4096
R = 512
C = 512

NC = 2
NS = 16
NW = NC * NS

BG = 8
BPG = B // BG
IC = NW // BG
NBR = R // 8
BRPC = NBR // IC

LANES = 16
JV = C // LANES


def _sc_gather(x, mask):
    mesh = plsc.VectorSubcoreMesh(core_axis_name="c", subcore_axis_name="s")

    @functools.partial(
        pl.kernel,
        mesh=mesh,
        compiler_params=pltpu.CompilerParams(needs_layout_passes=False),
        out_type=jax.ShapeDtypeStruct((B, R, C), jnp.float32),
        scratch_types=[
            pltpu.VMEM((BPG * V,), jnp.float32),
            pltpu.VMEM((8, C), jnp.int32),
            pltpu.VMEM((8, C), jnp.int32),
            pltpu.VMEM((BPG, 8, C), jnp.float32),
            pltpu.VMEM((BPG, 8, C), jnp.float32),
            pltpu.SemaphoreType.DMA,
            pltpu.SemaphoreType.DMA,
            pltpu.SemaphoreType.DMA,
            pltpu.SemaphoreType.DMA,
        ],
    )
    def k(x_hbm, mask_hbm, out_hbm, x_v, idx0, idx1, out0, out1,
          si0, si1, so0, so1):
        wid = lax.axis_index("s") * NC + lax.axis_index("c")
        g = wid // IC
        c = wid % IC
        row0 = g * BPG
        br0 = c * BRPC

        idxb = (idx0, idx1)
        outb = (out0, out1)
        sib = (si0, si1)
        sob = (so0, so1)

        def in_copy(s, par):
            return pltpu.make_async_copy(
                mask_hbm.at[pl.ds((br0 + s) * 8, 8)], idxb[par], sib[par])

        def out_copy(s, par, b):
            return pltpu.make_async_copy(
                outb[par].at[b],
                out_hbm.at[row0 + b, pl.ds((br0 + s) * 8, 8)],
                sob[par])

        in_copy(0, 0).start()
        in_copy(1, 1).start()

        for r in range(BPG):
            pltpu.sync_copy(x_hbm.at[row0 + r], x_v.at[pl.ds(r * V, V)])

        def step2(i, carry):
            for par in range(2):
                s = 2 * i + par
                in_copy(s, par).wait()

                @pl.when(s >= 2)
                def _drain():
                    for b in range(BPG):
                        out_copy(0, par, b).wait()

                @plsc.parallel_loop(0, 8 * JV, unroll=4)
                def _vbody(v):
                    ii = v >> 5
                    j0 = (v & 31) * LANES
                    idx = idxb[par][ii, pl.ds(j0, LANES)]
                    for b in range(BPG):
                        outb[par][b, ii, pl.ds(j0, LANES)] = (
                            plsc.load_gather(x_v, [idx + b * V]))

                @pl.when(s + 2 < BRPC)
                def _prefetch():
                    in_copy(s + 2, par).start()

                for b in range(BPG):
                    out_copy(s, par, b).start()
            return carry

        lax.fori_loop(0, BRPC // 2, step2, 0)
        for par in range(2):
            for b in range(BPG):
                out_copy(0, par, b).wait()

    return k(x, mask)


def kernel(x, mask):
    return _sc_gather(x, mask)


# speedup vs baseline: 15.3429x; 1.0014x over previous
import functools

import jax
import jax.numpy as jnp
from jax import lax
from jax.experimental import pallas as pl
from jax.experimental.pallas import tpu as pltpu
from jax.experimental.pallas import tpu_sc as plsc

B = 64
V = 4096
R = 512
C = 512

NC = 2
NS = 16
NW = NC * NS

BG = 8
BPG = B // BG
IC = NW // BG
NBR = R // 8
BRPC = NBR // IC

LANES = 16
JV = C // LANES


def _sc_gather(x, mask):
    mesh = plsc.VectorSubcoreMesh(core_axis_name="c", subcore_axis_name="s")

    @functools.partial(
        pl.kernel,
        mesh=mesh,
        compiler_params=pltpu.CompilerParams(
            needs_layout_passes=False, skip_device_barrier=True),
        out_type=jax.ShapeDtypeStruct((B, R, C), jnp.float32),
        scratch_types=[
            pltpu.VMEM((BPG * V,), jnp.float32),
            pltpu.VMEM((8, C), jnp.int32),
            pltpu.VMEM((8, C), jnp.int32),
            pltpu.VMEM((BPG, 8, C), jnp.float32),
            pltpu.VMEM((BPG, 8, C), jnp.float32),
            pltpu.SemaphoreType.DMA,
            pltpu.SemaphoreType.DMA,
            pltpu.SemaphoreType.DMA,
            pltpu.SemaphoreType.DMA,
        ],
    )
    def k(x_hbm, mask_hbm, out_hbm, x_v, idx0, idx1, out0, out1,
          si0, si1, so0, so1):
        wid = lax.axis_index("s") * NC + lax.axis_index("c")
        g = wid // IC
        c = wid % IC
        row0 = g * BPG
        br0 = c * BRPC

        idxb = (idx0, idx1)
        outb = (out0, out1)
        sib = (si0, si1)
        sob = (so0, so1)

        def in_copy(s, par):
            return pltpu.make_async_copy(
                mask_hbm.at[pl.ds((br0 + s) * 8, 8)], idxb[par], sib[par])

        def out_copy(s, par, b):
            return pltpu.make_async_copy(
                outb[par].at[b],
                out_hbm.at[row0 + b, pl.ds((br0 + s) * 8, 8)],
                sob[par])

        in_copy(0, 0).start()
        in_copy(1, 1).start()

        for r in range(BPG):
            pltpu.sync_copy(x_hbm.at[row0 + r], x_v.at[pl.ds(r * V, V)])

        def step2(i, carry):
            for par in range(2):
                s = 2 * i + par
                in_copy(s, par).wait()

                @pl.when(s >= 2)
                def _drain():
                    for b in range(BPG):
                        out_copy(0, par, b).wait()

                @plsc.parallel_loop(0, 8 * JV, unroll=4)
                def _vbody(v):
                    ii = v >> 5
                    j0 = (v & 31) * LANES
                    idx = idxb[par][ii, pl.ds(j0, LANES)]
                    for b in range(BPG):
                        outb[par][b, ii, pl.ds(j0, LANES)] = (
                            plsc.load_gather(x_v, [idx + b * V]))

                @pl.when(s + 2 < BRPC)
                def _prefetch():
                    in_copy(s + 2, par).start()

                for b in range(BPG):
                    out_copy(s, par, b).start()
            return carry

        lax.fori_loop(0, BRPC // 2, step2, 0)
        for par in range(2):
            for b in range(BPG):
                out_copy(0, par, b).wait()

    return k(x, mask)


def kernel(x, mask):
    return _sc_gather(x, mask)


# speedup vs baseline: 16.1686x; 1.0538x over previous
import functools

import jax
import jax.numpy as jnp
from jax import lax
from jax.experimental import pallas as pl
from jax.experimental.pallas import tpu as pltpu
from jax.experimental.pallas import tpu_sc as plsc

B = 64
V = 4096
R = 512
C = 512

NC = 2
NS = 16
NW = NC * NS

BG = 8
BPG = B // BG
IC = NW // BG
NBR = R // 8
BRPC = NBR // IC

LANES = 16
JV = C // LANES


def _sc_gather(x, mask):
    mesh = plsc.VectorSubcoreMesh(core_axis_name="c", subcore_axis_name="s")

    @functools.partial(
        pl.kernel,
        mesh=mesh,
        compiler_params=pltpu.CompilerParams(needs_layout_passes=False),
        out_type=jax.ShapeDtypeStruct((B, R, C), jnp.float32),
        scratch_types=[
            pltpu.VMEM((BPG * V,), jnp.float32),
            pltpu.VMEM((8, C), jnp.int32),
            pltpu.VMEM((8, C), jnp.int32),
            pltpu.VMEM((BPG, 8, C), jnp.float32),
            pltpu.VMEM((BPG, 8, C), jnp.float32),
            pltpu.SemaphoreType.DMA,
            pltpu.SemaphoreType.DMA,
            pltpu.SemaphoreType.DMA,
            pltpu.SemaphoreType.DMA,
        ],
    )
    def k(x_hbm, mask_hbm, out_hbm, x_v, idx0, idx1, out0, out1,
          si0, si1, so0, so1):
        wid = lax.axis_index("s") * NC + lax.axis_index("c")
        g = wid // IC
        c = wid % IC
        row0 = g * BPG
        br0 = c * BRPC

        idxb = (idx0, idx1)
        outb = (out0, out1)
        sib = (si0, si1)
        sob = (so0, so1)

        def in_copy(s, par):
            return pltpu.make_async_copy(
                mask_hbm.at[pl.ds((br0 + s) * 8, 8)], idxb[par], sib[par])

        def out_copy(s, par, b):
            return pltpu.make_async_copy(
                outb[par].at[b],
                out_hbm.at[row0 + b, pl.ds((br0 + s) * 8, 8)],
                sob[par])

        in_copy(0, 0).start()
        in_copy(1, 1).start()

        def x_copy(r):
            return pltpu.make_async_copy(
                x_hbm.at[row0 + r], x_v.at[pl.ds(r * V, V)], so0)

        for r in range(BPG):
            x_copy(r).start()
        for r in range(BPG):
            x_copy(r).wait()

        def step2(i, carry):
            for par in range(2):
                s = 2 * i + par
                in_copy(s, par).wait()

                @pl.when(s >= 2)
                def _drain():
                    for b in range(BPG):
                        out_copy(0, par, b).wait()

                @plsc.parallel_loop(0, 8 * JV, unroll=4)
                def _vbody(v):
                    ii = v >> 5
                    j0 = (v & 31) * LANES
                    idx = idxb[par][ii, pl.ds(j0, LANES)]
                    for b in range(BPG):
                        outb[par][b, ii, pl.ds(j0, LANES)] = (
                            plsc.load_gather(x_v, [idx + b * V]))

                @pl.when(s + 2 < BRPC)
                def _prefetch():
                    in_copy(s + 2, par).start()

                for b in range(BPG):
                    out_copy(s, par, b).start()
            return carry

        lax.fori_loop(0, BRPC // 2, step2, 0)
        for par in range(2):
            for b in range(BPG):
                out_copy(0, par, b).wait()

    return k(x, mask)


def kernel(x, mask):
    return _sc_gather(x, mask)


# speedup vs baseline: 16.2285x; 1.0037x over previous
import functools

import jax
import jax.numpy as jnp
from jax import lax
from jax.experimental import pallas as pl
from jax.experimental.pallas import tpu as pltpu
from jax.experimental.pallas import tpu_sc as plsc

B = 64
V = 4096
R = 512
C = 512

NC = 2
NS = 16
NW = NC * NS

BG = 8
BPG = B // BG
IC = NW // BG
NBR = R // 8
BRPC = NBR // IC

LANES = 16
JV = C // LANES


def _sc_gather(x, mask):
    mesh = plsc.VectorSubcoreMesh(core_axis_name="c", subcore_axis_name="s")

    @functools.partial(
        pl.kernel,
        mesh=mesh,
        compiler_params=pltpu.CompilerParams(needs_layout_passes=False),
        out_type=jax.ShapeDtypeStruct((B, R, C), jnp.float32),
        scratch_types=[
            pltpu.VMEM((BPG * V,), jnp.float32),
            pltpu.VMEM((8, C), jnp.int32),
            pltpu.VMEM((8, C), jnp.int32),
            pltpu.VMEM((BPG, 8, C), jnp.float32),
            pltpu.VMEM((BPG, 8, C), jnp.float32),
            pltpu.SemaphoreType.DMA,
            pltpu.SemaphoreType.DMA,
            pltpu.SemaphoreType.DMA,
            pltpu.SemaphoreType.DMA,
        ],
    )
    def k(x_hbm, mask_hbm, out_hbm, x_v, idx0, idx1, out0, out1,
          si0, si1, so0, so1):
        wid = lax.axis_index("s") * NC + lax.axis_index("c")
        g = wid // IC
        c = wid % IC
        row0 = g * BPG
        br0 = c * BRPC

        idxb = (idx0, idx1)
        outb = (out0, out1)
        sib = (si0, si1)
        sob = (so0, so1)

        def in_copy(s, par):
            return pltpu.make_async_copy(
                mask_hbm.at[pl.ds((br0 + s) * 8, 8)], idxb[par], sib[par])

        def out_copy(s, par, b):
            return pltpu.make_async_copy(
                outb[par].at[b],
                out_hbm.at[row0 + b, pl.ds((br0 + s) * 8, 8)],
                sob[par])

        in_copy(0, 0).start()
        in_copy(1, 1).start()

        def x_copy(r):
            return pltpu.make_async_copy(
                x_hbm.at[row0 + r], x_v.at[pl.ds(r * V, V)], so0)

        for r in range(BPG):
            x_copy(r).start()
        for r in range(BPG):
            x_copy(r).wait()

        def step2(i, carry):
            for par in range(2):
                s = 2 * i + par
                in_copy(s, par).wait()

                @pl.when(s >= 2)
                def _drain():
                    for b in range(BPG):
                        out_copy(0, par, b).wait()

                @plsc.parallel_loop(0, 8 * JV, unroll=2)
                def _vbody(v):
                    ii = v >> 5
                    j0 = (v & 31) * LANES
                    idx = idxb[par][ii, pl.ds(j0, LANES)]
                    for b in range(BPG):
                        outb[par][b, ii, pl.ds(j0, LANES)] = (
                            plsc.load_gather(x_v, [idx + b * V]))

                @pl.when(s + 2 < BRPC)
                def _prefetch():
                    in_copy(s + 2, par).start()

                for b in range(BPG):
                    out_copy(s, par, b).start()
            return carry

        lax.fori_loop(0, BRPC // 2, step2, 0)
        for par in range(2):
            for b in range(BPG):
                out_copy(0, par, b).wait()

    return k(x, mask)


def kernel(x, mask):
    return _sc_gather(x, mask)
